# baseline (speedup 1.0000x reference)
import functools

import jax
import jax.numpy as jnp
from jax import lax
from jax.experimental import pallas as pl
from jax.experimental.pallas import tpu as pltpu
from jax.experimental.pallas import tpu_sc as plsc

_D = 64
_NEG = 5
_K = 1 + _NEG
_B = 16384
_V = 1000000
_V2 = _V // 2
_L = 16
_NC = 2
_NS = 16
_NW = _NC * _NS
_BW = _B // _NW
_CHUNK = 64
_NCHUNK = _BW // _CHUNK

_mesh = plsc.VectorSubcoreMesh(
    core_axis_name="c", subcore_axis_name="s",
    num_cores=_NC, num_subcores=_NS)


_UBLK = 10000
_UGRID = _V // _UBLK


def _untile_body(x_ref, o_ref):
    x = x_ref[...].reshape(_UBLK // 2, 2, _D)
    o_ref[...] = jnp.concatenate([x[:, 0, :], x[:, 1, :]], axis=1)


_tc_untile = pl.pallas_call(
    _untile_body,
    grid=(_UGRID,),
    in_specs=[pl.BlockSpec((_UBLK, _D), lambda i: (i, 0))],
    out_specs=pl.BlockSpec((_UBLK // 2, 2 * _D), lambda i: (i, 0)),
    out_shape=jax.ShapeDtypeStruct((_V2, 2 * _D), jnp.float32),
)


@functools.partial(
    pl.kernel,
    out_type=jax.ShapeDtypeStruct((_B, 2 * _D), jnp.float32),
    mesh=_mesh,
    compiler_params=pltpu.CompilerParams(use_tc_tiling_on_sc=True),
    scratch_types=[
        pltpu.VMEM((_BW,), jnp.int32),
        pltpu.VMEM((8, _BW), jnp.int32),
        pltpu.VMEM((_BW + _L,), jnp.int32),
        pltpu.VMEM((_CHUNK, 2 * _D), jnp.float32),
        pltpu.VMEM((_K, _CHUNK, 2 * _D), jnp.float32),
        pltpu.VMEM((_CHUNK, 2 * _D), jnp.float32),
        pltpu.SemaphoreType.DMA,
    ],
)
def _sc_partials(idx_w, idx_cn, par, emb0, emb1, out,
                 idx_w_v, idx_cn_v, par_v, w_v, cn_v, t_v, sem):
    wid = lax.axis_index("s") * _NC + lax.axis_index("c")
    base = wid * _BW
    pltpu.sync_copy(idx_w.at[pl.ds(base, _BW)], idx_w_v)
    pltpu.sync_copy(idx_cn.at[:, pl.ds(base, _BW)], idx_cn_v)
    pltpu.sync_copy(par.at[pl.ds(base, _BW + _L)], par_v)

    def chunk(ci, carry):
        off = pl.multiple_of(ci * _CHUNK, _CHUNK)
        cps = [pltpu.async_copy(
            emb0.at[idx_w_v.at[pl.ds(off, _CHUNK)]], w_v, sem)]
        for j in range(_K):
            cps.append(pltpu.async_copy(
                emb1.at[idx_cn_v.at[j, pl.ds(off, _CHUNK)]], cn_v.at[j], sem))
        for cp in cps:
            cp.wait()

        def group(g, c2):
            pv = par_v[pl.ds(off + g * _L, _L)]
            for i2 in range(_L):
                i = g * _L + i2
                p = pv[i2]
                ow = (p & 1) * _D
                w0 = w_v[i, pl.ds(ow, _L)]
                w1 = w_v[i, pl.ds(ow + _L, _L)]
                w2 = w_v[i, pl.ds(ow + 2 * _L, _L)]
                w3 = w_v[i, pl.ds(ow + 3 * _L, _L)]
                for j in range(_K):
                    oj = ((p >> (j + 1)) & 1) * _D
                    t = (w0 * cn_v[j, i, pl.ds(oj, _L)]
                         + w1 * cn_v[j, i, pl.ds(oj + _L, _L)]
                         + w2 * cn_v[j, i, pl.ds(oj + 2 * _L, _L)]
                         + w3 * cn_v[j, i, pl.ds(oj + 3 * _L, _L)])
                    t_v[i, pl.ds(j * _L, _L)] = t
            return c2

        lax.fori_loop(0, _CHUNK // _L, group, 0)
        pltpu.sync_copy(t_v, out.at[pl.ds(base + off, _CHUNK)])
        return carry

    lax.fori_loop(0, _NCHUNK, chunk, 0)


_TC_BLK = 2048
_TC_GRID = _B // _TC_BLK


def _tc_loss(t_ref, mask_ref, pos_ref, neg_ref):
    step = pl.program_id(0)
    x = t_ref[:, 0:_K * _L]
    m = mask_ref[...]
    pos_ip = jnp.sum(x[:, 0:_L], axis=1, keepdims=True)
    pos_part = jnp.sum(-jax.nn.log_sigmoid(pos_ip))
    neg_part = jnp.float32(0.0)
    for j in range(1, _K):
        ip = jnp.sum(x[:, j * _L:(j + 1) * _L], axis=1, keepdims=True)
        ip = ip * m[:, j - 1:j]
        neg_part = neg_part + jnp.sum(-jax.nn.log_sigmoid(-ip))

    @pl.when(step == 0)
    def _():
        pos_ref[0, 0] = jnp.float32(0.0)
        neg_ref[0, 0] = jnp.float32(0.0)

    pos_ref[0, 0] += pos_part
    neg_ref[0, 0] += neg_part


_tc_call = pl.pallas_call(
    _tc_loss,
    grid=(_TC_GRID,),
    in_specs=[
        pl.BlockSpec((_TC_BLK, 2 * _D), lambda i: (i, 0)),
        pl.BlockSpec((_TC_BLK, _NEG), lambda i: (i, 0)),
    ],
    out_specs=(
        pl.BlockSpec((1, 1), lambda i: (0, 0), memory_space=pltpu.SMEM),
        pl.BlockSpec((1, 1), lambda i: (0, 0), memory_space=pltpu.SMEM),
    ),
    out_shape=(jax.ShapeDtypeStruct((1, 1), jnp.float32),
               jax.ShapeDtypeStruct((1, 1), jnp.float32)),
)


def kernel(data, emb0, emb1):
    idx = data[:, 0:1 + _K].astype(jnp.int32)
    idx_w = idx[:, 0] >> 1
    idx_cn = jnp.concatenate(
        [idx[:, 1:].T >> 1, jnp.zeros((2, _B), jnp.int32)], axis=0)
    par = jnp.sum((idx & 1) << jnp.arange(1 + _K, dtype=jnp.int32)[None, :],
                  axis=1, dtype=jnp.int32)
    par = jnp.concatenate([par, jnp.zeros((_L,), jnp.int32)])
    mask = data[:, 1 + _K:].astype(jnp.float32)
    emb0r = _tc_untile(emb0)
    emb1r = _tc_untile(emb1)
    t = _sc_partials(idx_w, idx_cn, par, emb0r, emb1r)
    pos, neg = _tc_call(t, mask)
    return (pos[0, 0], neg[0, 0])


# speedup vs baseline: 1.2790x; 1.2790x over previous
import functools

import jax
import jax.numpy as jnp
from jax import lax
from jax.experimental import pallas as pl
from jax.experimental.pallas import tpu as pltpu
from jax.experimental.pallas import tpu_sc as plsc

_D = 64
_NEG = 5
_K = 1 + _NEG
_B = 16384
_L = 16
_NC = 2
_NS = 16
_NW = _NC * _NS
_BW = _B // _NW
_CHUNK = 128
_NCHUNK = _BW // _CHUNK

_mesh = plsc.VectorSubcoreMesh(
    core_axis_name="c", subcore_axis_name="s",
    num_cores=_NC, num_subcores=_NS)


@functools.partial(
    pl.kernel,
    out_type=jax.ShapeDtypeStruct((_B, _K * _L), jnp.float32),
    mesh=_mesh,
    compiler_params=pltpu.CompilerParams(use_tc_tiling_on_sc=False),
    scratch_types=[
        pltpu.VMEM((_BW,), jnp.int32),
        pltpu.VMEM((_K, _BW), jnp.int32),
        pltpu.VMEM((_CHUNK, _D), jnp.float32),
        pltpu.VMEM((_K, _CHUNK, _D), jnp.float32),
        pltpu.VMEM((_CHUNK, _K * _L), jnp.float32),
        pltpu.SemaphoreType.DMA,
    ],
)
def _sc_partials(idx_w, idx_cn, emb0, emb1, out,
                 idx_w_v, idx_cn_v, w_v, cn_v, t_v, sem):
    wid = lax.axis_index("s") * _NC + lax.axis_index("c")
    base = wid * _BW
    pltpu.sync_copy(idx_w.at[pl.ds(base, _BW)], idx_w_v)
    pltpu.sync_copy(idx_cn.at[:, pl.ds(base, _BW)], idx_cn_v)

    def chunk(ci, carry):
        off = pl.multiple_of(ci * _CHUNK, _CHUNK)
        cps = [pltpu.async_copy(
            emb0.at[idx_w_v.at[pl.ds(off, _CHUNK)]], w_v, sem)]
        for j in range(_K):
            cps.append(pltpu.async_copy(
                emb1.at[idx_cn_v.at[j, pl.ds(off, _CHUNK)]], cn_v.at[j], sem))
        for cp in cps:
            cp.wait()

        def elem(i, c2):
            w0 = w_v[i, pl.ds(0, _L)]
            w1 = w_v[i, pl.ds(_L, _L)]
            w2 = w_v[i, pl.ds(2 * _L, _L)]
            w3 = w_v[i, pl.ds(3 * _L, _L)]
            for j in range(_K):
                t = (w0 * cn_v[j, i, pl.ds(0, _L)]
                     + w1 * cn_v[j, i, pl.ds(_L, _L)]
                     + w2 * cn_v[j, i, pl.ds(2 * _L, _L)]
                     + w3 * cn_v[j, i, pl.ds(3 * _L, _L)])
                t_v[i, pl.ds(j * _L, _L)] = t
            return c2

        lax.fori_loop(0, _CHUNK, elem, 0)
        pltpu.sync_copy(t_v, out.at[pl.ds(base + off, _CHUNK)])
        return carry

    lax.fori_loop(0, _NCHUNK, chunk, 0)


_TC_BLK = 2048
_TC_GRID = _B // _TC_BLK


def _tc_loss(t_ref, mask_ref, pos_ref, neg_ref):
    step = pl.program_id(0)
    x = t_ref[...]
    m = mask_ref[...]
    cols = [jnp.sum(x[:, 0:_L], axis=1, keepdims=True)]
    for j in range(1, _K):
        ip = jnp.sum(x[:, j * _L:(j + 1) * _L], axis=1, keepdims=True)
        cols.append(-ip * m[:, j - 1:j])
    y = jnp.concatenate(cols, axis=1)
    terms = -jax.nn.log_sigmoid(y)
    pos_part = jnp.sum(terms[:, 0:1])
    neg_part = jnp.sum(terms[:, 1:])

    @pl.when(step == 0)
    def _():
        pos_ref[0, 0] = jnp.float32(0.0)
        neg_ref[0, 0] = jnp.float32(0.0)

    pos_ref[0, 0] += pos_part
    neg_ref[0, 0] += neg_part


_tc_call = pl.pallas_call(
    _tc_loss,
    grid=(_TC_GRID,),
    in_specs=[
        pl.BlockSpec((_TC_BLK, _K * _L), lambda i: (i, 0)),
        pl.BlockSpec((_TC_BLK, _NEG), lambda i: (i, 0)),
    ],
    out_specs=(
        pl.BlockSpec((1, 1), lambda i: (0, 0), memory_space=pltpu.SMEM),
        pl.BlockSpec((1, 1), lambda i: (0, 0), memory_space=pltpu.SMEM),
    ),
    out_shape=(jax.ShapeDtypeStruct((1, 1), jnp.float32),
               jax.ShapeDtypeStruct((1, 1), jnp.float32)),
)


def kernel(data, emb0, emb1):
    idx = data[:, 0:1 + _K].astype(jnp.int32)
    idx_w = idx[:, 0]
    idx_cn = idx[:, 1:].T
    mask = data[:, 1 + _K:].astype(jnp.float32)
    t = _sc_partials(idx_w, idx_cn, emb0, emb1)
    pos, neg = _tc_call(t, mask)
    return (pos[0, 0], neg[0, 0])
